# baseline (speedup 1.0000x reference)
import jax
import jax.numpy as jnp
from jax import lax
from jax.experimental import pallas as pl
from jax.experimental.pallas import tpu as pltpu
from jax.experimental.pallas import tpu_sc as plsc

N = 10000
D = 256
DH = 128
E = 64000
NTILES = 16
CHUNK = 50
NCHUNK = (E // NTILES) // CHUNK
NMAIN = (NCHUNK - 2) // 3 * 3
ROWS_PT = N // NTILES
ZROWS = 25
ZCROWS = 125
CW = 16
NR = 3


def _sc_body(featL, featR, srcs, dsts, aggL, aggR, cnt,
             agg_sh, cnt_sh, src_v, dst_v, rows_a, rows_b, rows_c, ones_v,
             zero_v, zcnt_v, sga, sgb, sgc, ssa, ssb, ssc, scnt, semz):
  c = lax.axis_index("c")
  t = lax.axis_index("s")

  @pl.loop(0, ZCROWS)
  def _(i):
    zcnt_v[i, :] = jnp.zeros((16,), jnp.float32)

  @pl.loop(0, CHUNK)
  def _(i):
    ones_v[i, :] = jnp.ones((16,), jnp.float32)

  @pl.loop(0, ZROWS)
  def _(i):
    for c16 in range(DH // 16):
      zero_v[i, pl.ds(c16 * 16, 16)] = jnp.zeros((16,), jnp.float32)

  def chunk_loop(feat_hbm, counts):
    def fire_cnt(j):
      if counts:
        pltpu.async_copy(ones_v, cnt_sh.at[dst_v.at[j]], scnt, add=True)

    pltpu.async_copy(feat_hbm.at[src_v.at[0]], rows_a, sga)
    pltpu.async_copy(feat_hbm.at[src_v.at[1]], rows_b, sgb)

    @pl.loop(0, NMAIN, step=3)
    def _(j):
      @pl.when(j > 0)
      def _():
        pltpu.make_async_copy(rows_c, agg_sh.at[dst_v.at[j - 1]], ssc).wait()
      pltpu.make_async_copy(feat_hbm.at[src_v.at[j]], rows_a, sga).wait()
      pltpu.async_copy(rows_a, agg_sh.at[dst_v.at[j]], ssa, add=True)
      fire_cnt(j)
      pltpu.async_copy(feat_hbm.at[src_v.at[j + 2]], rows_c, sgc)
      pltpu.make_async_copy(feat_hbm.at[src_v.at[j + 1]], rows_b, sgb).wait()
      pltpu.async_copy(rows_b, agg_sh.at[dst_v.at[j + 1]], ssb, add=True)
      fire_cnt(j + 1)
      pltpu.make_async_copy(rows_a, agg_sh.at[dst_v.at[j]], ssa).wait()
      pltpu.async_copy(feat_hbm.at[src_v.at[j + 3]], rows_a, sga)
      pltpu.make_async_copy(feat_hbm.at[src_v.at[j + 2]], rows_c, sgc).wait()
      pltpu.async_copy(rows_c, agg_sh.at[dst_v.at[j + 2]], ssc, add=True)
      fire_cnt(j + 2)
      pltpu.make_async_copy(rows_b, agg_sh.at[dst_v.at[j + 1]], ssb).wait()
      pltpu.async_copy(feat_hbm.at[src_v.at[j + 4]], rows_b, sgb)

    pltpu.make_async_copy(rows_c, agg_sh.at[dst_v.at[NMAIN - 1]], ssc).wait()
    pltpu.make_async_copy(feat_hbm.at[src_v.at[NMAIN]], rows_a, sga).wait()
    pltpu.async_copy(rows_a, agg_sh.at[dst_v.at[NMAIN]], ssa, add=True)
    fire_cnt(NMAIN)
    pltpu.make_async_copy(feat_hbm.at[src_v.at[NMAIN + 1]], rows_b, sgb).wait()
    pltpu.async_copy(rows_b, agg_sh.at[dst_v.at[NMAIN + 1]], ssb, add=True)
    fire_cnt(NMAIN + 1)
    pltpu.make_async_copy(rows_a, agg_sh.at[dst_v.at[NMAIN]], ssa).wait()
    pltpu.make_async_copy(rows_b, agg_sh.at[dst_v.at[NMAIN + 1]], ssb).wait()

    if counts:
      @pl.loop(0, NCHUNK)
      def _(j):
        pltpu.make_async_copy(ones_v, cnt_sh.at[dst_v.at[j]], scnt).wait()

  @pl.loop(0, NR)
  def _(r):
    @pl.loop(0, ROWS_PT // ZROWS)
    def _(i):
      pltpu.async_copy(
          zero_v, agg_sh.at[pl.ds(t * ROWS_PT + i * ZROWS, ZROWS)], semz)

    @pl.when(c == 0)
    def _():
      @pl.loop(0, ROWS_PT // ZCROWS)
      def _(i):
        pltpu.async_copy(
            zcnt_v, cnt_sh.at[pl.ds(t * ROWS_PT + i * ZCROWS, ZCROWS)], semz)

    @pl.loop(0, ROWS_PT // ZROWS)
    def _(i):
      pltpu.make_async_copy(
          zero_v, agg_sh.at[pl.ds(t * ROWS_PT + i * ZROWS, ZROWS)], semz).wait()

    @pl.when(c == 0)
    def _():
      @pl.loop(0, ROWS_PT // ZCROWS)
      def _(i):
        pltpu.make_async_copy(
            zcnt_v, cnt_sh.at[pl.ds(t * ROWS_PT + i * ZCROWS, ZCROWS)],
            semz).wait()

    pltpu.sync_copy(srcs.at[r, t], src_v)
    pltpu.sync_copy(dsts.at[r, t], dst_v)

    plsc.subcore_barrier()

    @pl.when(c == 0)
    def _():
      chunk_loop(featL, True)

    @pl.when(c == 1)
    def _():
      chunk_loop(featR, False)

    plsc.subcore_barrier()

    rows = pl.ds(t * ROWS_PT, ROWS_PT)

    @pl.when(c == 0)
    def _():
      pltpu.sync_copy(agg_sh.at[rows], aggL.at[r].at[rows])
      pltpu.sync_copy(cnt_sh.at[rows], cnt.at[r].at[rows])

    @pl.when(c == 1)
    def _():
      pltpu.sync_copy(agg_sh.at[rows], aggR.at[r].at[rows])


@jax.jit
def _sc_aggregate(featL, featR, srcs, dsts):
  out = [jax.ShapeDtypeStruct((NR, N, DH), jnp.float32),
         jax.ShapeDtypeStruct((NR, N, DH), jnp.float32),
         jax.ShapeDtypeStruct((NR, N, CW), jnp.float32)]
  scratch = [
      pltpu.MemorySpace.VMEM_SHARED((N, DH), jnp.float32),
      pltpu.MemorySpace.VMEM_SHARED((N, CW), jnp.float32),
      pltpu.MemorySpace.VMEM((NCHUNK, CHUNK), jnp.int32),
      pltpu.MemorySpace.VMEM((NCHUNK, CHUNK), jnp.int32),
      pltpu.MemorySpace.VMEM((CHUNK, DH), jnp.float32),
      pltpu.MemorySpace.VMEM((CHUNK, DH), jnp.float32),
      pltpu.MemorySpace.VMEM((CHUNK, DH), jnp.float32),
      pltpu.MemorySpace.VMEM((CHUNK, CW), jnp.float32),
      pltpu.MemorySpace.VMEM((ZROWS, DH), jnp.float32),
      pltpu.MemorySpace.VMEM((ZCROWS, CW), jnp.float32),
      pltpu.SemaphoreType.DMA,
      pltpu.SemaphoreType.DMA,
      pltpu.SemaphoreType.DMA,
      pltpu.SemaphoreType.DMA,
      pltpu.SemaphoreType.DMA,
      pltpu.SemaphoreType.DMA,
      pltpu.SemaphoreType.DMA,
      pltpu.SemaphoreType.DMA,
  ]
  mesh = plsc.VectorSubcoreMesh(core_axis_name="c", subcore_axis_name="s",
                                num_cores=2, num_subcores=16)
  return pl.kernel(
      _sc_body, out_type=out, mesh=mesh, scratch_types=scratch,
      compiler_params=pltpu.CompilerParams(use_tc_tiling_on_sc=False))(
      featL, featR, srcs, dsts)


BN = 1000


def _tc_body(feat_b, aL0, aR0, c0, aL1, aR1, c1, aL2, aR2, c2,
             W0b, WT0, WB0, WT1, WB1, WT2, WB2, b0b, br0, br1, br2, out):
  bf16 = jnp.bfloat16
  acc = jnp.dot(feat_b[...].astype(bf16), W0b[...],
                preferred_element_type=jnp.float32) + b0b[...]
  for aL, aR, cn, WT, WB, br in (
      (aL0, aR0, c0, WT0, WB0, br0),
      (aL1, aR1, c1, WT1, WB1, br1),
      (aL2, aR2, c2, WT2, WB2, br2),
  ):
    cnt = cn[0, :, 0:1]
    inv = 1.0 / jnp.maximum(cnt, 1.0)
    acc += jnp.dot((aL[0] * inv).astype(bf16), WT[...],
                   preferred_element_type=jnp.float32)
    acc += jnp.dot((aR[0] * inv).astype(bf16), WB[...],
                   preferred_element_type=jnp.float32)
    acc += jnp.where(cnt > 0.0, 1.0, 0.0) * br[...]
  out[...] = acc


@jax.jit
def _tc_combine(feat, aggL, aggR, cnt, W0, WT0, WB0, WT1, WB1, WT2, WB2,
                b0, br0, br1, br2):
  full = lambda a: pl.BlockSpec(a.shape, lambda i: (0, 0))
  in_specs = [pl.BlockSpec((BN, D), lambda i: (i, 0))]
  args = [feat]
  for r in range(NR):
    in_specs += [pl.BlockSpec((1, BN, DH), lambda i, r=r: (r, i, 0)),
                 pl.BlockSpec((1, BN, DH), lambda i, r=r: (r, i, 0)),
                 pl.BlockSpec((1, BN, CW), lambda i, r=r: (r, i, 0))]
    args += [aggL, aggR, cnt]
  in_specs += [full(W0), full(WT0), full(WB0), full(WT1), full(WB1),
               full(WT2), full(WB2), full(b0), full(br0), full(br1), full(br2)]
  args += [W0, WT0, WB0, WT1, WB1, WT2, WB2, b0, br0, br1, br2]
  return pl.pallas_call(
      _tc_body,
      grid=(N // BN,),
      in_specs=in_specs,
      out_specs=pl.BlockSpec((BN, D), lambda i: (i, 0)),
      out_shape=jax.ShapeDtypeStruct((N, D), jnp.float32),
  )(*args)


def kernel(feat, edge_index_r0, edge_index_r1, edge_index_r2,
           W0, b0, W_r0, b_r0, W_r1, b_r1, W_r2, b_r2):
  featL = feat[:, :DH]
  featR = feat[:, DH:]
  ei = jnp.stack([edge_index_r0, edge_index_r1, edge_index_r2])
  ei = ei.reshape(NR, 2, NTILES, NCHUNK, CHUNK)
  srcs = ei[:, 0]
  dsts = ei[:, 1]

  aggL, aggR, cnt = _sc_aggregate(featL, featR, srcs, dsts)

  bf16 = jnp.bfloat16
  return _tc_combine(
      feat, aggL, aggR, cnt,
      W0.astype(bf16), W_r0[:DH].astype(bf16), W_r0[DH:].astype(bf16),
      W_r1[:DH].astype(bf16), W_r1[DH:].astype(bf16),
      W_r2[:DH].astype(bf16), W_r2[DH:].astype(bf16),
      b0.reshape(1, D), b_r0.reshape(1, D), b_r1.reshape(1, D),
      b_r2.reshape(1, D))


# speedup vs baseline: 1.0235x; 1.0235x over previous
import jax
import jax.numpy as jnp
from jax import lax
from jax.experimental import pallas as pl
from jax.experimental.pallas import tpu as pltpu
from jax.experimental.pallas import tpu_sc as plsc

N = 10000
D = 256
DH = 128
E = 64000
NTILES = 16
CHUNK = 50
NCHUNK = (E // NTILES) // CHUNK
NMAIN = (NCHUNK - 2) // 3 * 3
ROWS_PT = N // NTILES
ZROWS = 25
ZCROWS = 125
CW = 16
NR = 3


def _sc_body(featL, featR, srcs, dsts, aggL, aggR, cnt,
             agg_sh, cnt_sh, src_v, dst_v, rows_a, rows_b, rows_c, ones_v,
             zero_v, zcnt_v, sga, sgb, sgc, ssa, ssb, ssc, scnt, semz):
  c = lax.axis_index("c")
  t = lax.axis_index("s")

  @pl.loop(0, ZCROWS)
  def _(i):
    zcnt_v[i, :] = jnp.zeros((16,), jnp.float32)

  @pl.loop(0, CHUNK)
  def _(i):
    ones_v[i, :] = jnp.ones((16,), jnp.float32)

  @pl.loop(0, ZROWS)
  def _(i):
    for c16 in range(DH // 16):
      zero_v[i, pl.ds(c16 * 16, 16)] = jnp.zeros((16,), jnp.float32)

  def chunk_loop(feat_hbm, counts):
    def fire_cnt(j):
      if counts:
        pltpu.async_copy(ones_v, cnt_sh.at[dst_v.at[j]], scnt, add=True)

    pltpu.async_copy(feat_hbm.at[src_v.at[0]], rows_a, sga)
    pltpu.async_copy(feat_hbm.at[src_v.at[1]], rows_b, sgb)

    @pl.loop(0, NMAIN, step=3)
    def _(j):
      @pl.when(j > 0)
      def _():
        pltpu.make_async_copy(rows_c, agg_sh.at[dst_v.at[j - 1]], ssc).wait()
      pltpu.make_async_copy(feat_hbm.at[src_v.at[j]], rows_a, sga).wait()
      pltpu.async_copy(rows_a, agg_sh.at[dst_v.at[j]], ssa, add=True)
      fire_cnt(j)
      pltpu.async_copy(feat_hbm.at[src_v.at[j + 2]], rows_c, sgc)
      pltpu.make_async_copy(feat_hbm.at[src_v.at[j + 1]], rows_b, sgb).wait()
      pltpu.async_copy(rows_b, agg_sh.at[dst_v.at[j + 1]], ssb, add=True)
      fire_cnt(j + 1)
      pltpu.make_async_copy(rows_a, agg_sh.at[dst_v.at[j]], ssa).wait()
      pltpu.async_copy(feat_hbm.at[src_v.at[j + 3]], rows_a, sga)
      pltpu.make_async_copy(feat_hbm.at[src_v.at[j + 2]], rows_c, sgc).wait()
      pltpu.async_copy(rows_c, agg_sh.at[dst_v.at[j + 2]], ssc, add=True)
      fire_cnt(j + 2)
      pltpu.make_async_copy(rows_b, agg_sh.at[dst_v.at[j + 1]], ssb).wait()
      pltpu.async_copy(feat_hbm.at[src_v.at[j + 4]], rows_b, sgb)

    pltpu.make_async_copy(rows_c, agg_sh.at[dst_v.at[NMAIN - 1]], ssc).wait()
    pltpu.make_async_copy(feat_hbm.at[src_v.at[NMAIN]], rows_a, sga).wait()
    pltpu.async_copy(rows_a, agg_sh.at[dst_v.at[NMAIN]], ssa, add=True)
    fire_cnt(NMAIN)
    pltpu.make_async_copy(feat_hbm.at[src_v.at[NMAIN + 1]], rows_b, sgb).wait()
    pltpu.async_copy(rows_b, agg_sh.at[dst_v.at[NMAIN + 1]], ssb, add=True)
    fire_cnt(NMAIN + 1)
    pltpu.make_async_copy(rows_a, agg_sh.at[dst_v.at[NMAIN]], ssa).wait()
    pltpu.make_async_copy(rows_b, agg_sh.at[dst_v.at[NMAIN + 1]], ssb).wait()

    if counts:
      @pl.loop(0, NCHUNK)
      def _(j):
        pltpu.make_async_copy(ones_v, cnt_sh.at[dst_v.at[j]], scnt).wait()

  @pl.loop(0, NR)
  def _(r):
    @pl.loop(0, ROWS_PT // ZROWS)
    def _(i):
      pltpu.async_copy(
          zero_v, agg_sh.at[pl.ds(t * ROWS_PT + i * ZROWS, ZROWS)], semz)

    @pl.when(c == 0)
    def _():
      @pl.loop(0, ROWS_PT // ZCROWS)
      def _(i):
        pltpu.async_copy(
            zcnt_v, cnt_sh.at[pl.ds(t * ROWS_PT + i * ZCROWS, ZCROWS)], semz)

    pltpu.sync_copy(srcs.at[r, t], src_v)
    pltpu.sync_copy(dsts.at[r, t], dst_v)

    @pl.loop(0, ROWS_PT // ZROWS)
    def _(i):
      pltpu.make_async_copy(
          zero_v, agg_sh.at[pl.ds(t * ROWS_PT + i * ZROWS, ZROWS)], semz).wait()

    @pl.when(c == 0)
    def _():
      @pl.loop(0, ROWS_PT // ZCROWS)
      def _(i):
        pltpu.make_async_copy(
            zcnt_v, cnt_sh.at[pl.ds(t * ROWS_PT + i * ZCROWS, ZCROWS)],
            semz).wait()

    plsc.subcore_barrier()

    @pl.when(c == 0)
    def _():
      chunk_loop(featL, True)

    @pl.when(c == 1)
    def _():
      chunk_loop(featR, False)

    plsc.subcore_barrier()

    rows = pl.ds(t * ROWS_PT, ROWS_PT)

    @pl.when(c == 0)
    def _():
      oa = pltpu.async_copy(agg_sh.at[rows], aggL.at[r].at[rows], sga)
      ob = pltpu.async_copy(cnt_sh.at[rows], cnt.at[r].at[rows], sgb)
      oa.wait()
      ob.wait()

    @pl.when(c == 1)
    def _():
      pltpu.sync_copy(agg_sh.at[rows], aggR.at[r].at[rows])


@jax.jit
def _sc_aggregate(featL, featR, srcs, dsts):
  out = [jax.ShapeDtypeStruct((NR, N, DH), jnp.float32),
         jax.ShapeDtypeStruct((NR, N, DH), jnp.float32),
         jax.ShapeDtypeStruct((NR, N, CW), jnp.float32)]
  scratch = [
      pltpu.MemorySpace.VMEM_SHARED((N, DH), jnp.float32),
      pltpu.MemorySpace.VMEM_SHARED((N, CW), jnp.float32),
      pltpu.MemorySpace.VMEM((NCHUNK, CHUNK), jnp.int32),
      pltpu.MemorySpace.VMEM((NCHUNK, CHUNK), jnp.int32),
      pltpu.MemorySpace.VMEM((CHUNK, DH), jnp.float32),
      pltpu.MemorySpace.VMEM((CHUNK, DH), jnp.float32),
      pltpu.MemorySpace.VMEM((CHUNK, DH), jnp.float32),
      pltpu.MemorySpace.VMEM((CHUNK, CW), jnp.float32),
      pltpu.MemorySpace.VMEM((ZROWS, DH), jnp.float32),
      pltpu.MemorySpace.VMEM((ZCROWS, CW), jnp.float32),
      pltpu.SemaphoreType.DMA,
      pltpu.SemaphoreType.DMA,
      pltpu.SemaphoreType.DMA,
      pltpu.SemaphoreType.DMA,
      pltpu.SemaphoreType.DMA,
      pltpu.SemaphoreType.DMA,
      pltpu.SemaphoreType.DMA,
      pltpu.SemaphoreType.DMA,
  ]
  mesh = plsc.VectorSubcoreMesh(core_axis_name="c", subcore_axis_name="s",
                                num_cores=2, num_subcores=16)
  return pl.kernel(
      _sc_body, out_type=out, mesh=mesh, scratch_types=scratch,
      compiler_params=pltpu.CompilerParams(use_tc_tiling_on_sc=False))(
      featL, featR, srcs, dsts)


BN = 1000


def _tc_body(feat_b, aL0, aR0, c0, aL1, aR1, c1, aL2, aR2, c2,
             W0b, WT0, WB0, WT1, WB1, WT2, WB2, b0b, br0, br1, br2, out):
  acc = jnp.dot(feat_b[...], W0b[...],
                preferred_element_type=jnp.float32) + b0b[...]
  for aL, aR, cn, WT, WB, br in (
      (aL0, aR0, c0, WT0, WB0, br0),
      (aL1, aR1, c1, WT1, WB1, br1),
      (aL2, aR2, c2, WT2, WB2, br2),
  ):
    cnt = cn[0, :, 0:1]
    inv = 1.0 / jnp.maximum(cnt, 1.0)
    acc += jnp.dot(aL[0] * inv, WT[...], preferred_element_type=jnp.float32)
    acc += jnp.dot(aR[0] * inv, WB[...], preferred_element_type=jnp.float32)
    acc += jnp.where(cnt > 0.0, 1.0, 0.0) * br[...]
  out[...] = acc


@jax.jit
def _tc_combine(feat, aggL, aggR, cnt, W0, WT0, WB0, WT1, WB1, WT2, WB2,
                b0, br0, br1, br2):
  full = lambda a: pl.BlockSpec(a.shape, lambda i: (0, 0))
  in_specs = [pl.BlockSpec((BN, D), lambda i: (i, 0))]
  args = [feat]
  for r in range(NR):
    in_specs += [pl.BlockSpec((1, BN, DH), lambda i, r=r: (r, i, 0)),
                 pl.BlockSpec((1, BN, DH), lambda i, r=r: (r, i, 0)),
                 pl.BlockSpec((1, BN, CW), lambda i, r=r: (r, i, 0))]
    args += [aggL, aggR, cnt]
  in_specs += [full(W0), full(WT0), full(WB0), full(WT1), full(WB1),
               full(WT2), full(WB2), full(b0), full(br0), full(br1), full(br2)]
  args += [W0, WT0, WB0, WT1, WB1, WT2, WB2, b0, br0, br1, br2]
  return pl.pallas_call(
      _tc_body,
      grid=(N // BN,),
      in_specs=in_specs,
      out_specs=pl.BlockSpec((BN, D), lambda i: (i, 0)),
      out_shape=jax.ShapeDtypeStruct((N, D), jnp.float32),
  )(*args)


def kernel(feat, edge_index_r0, edge_index_r1, edge_index_r2,
           W0, b0, W_r0, b_r0, W_r1, b_r1, W_r2, b_r2):
  featL = feat[:, :DH]
  featR = feat[:, DH:]
  ei = jnp.stack([edge_index_r0, edge_index_r1, edge_index_r2])
  ei = ei.reshape(NR, 2, NTILES, NCHUNK, CHUNK)
  srcs = ei[:, 0]
  dsts = ei[:, 1]

  aggL, aggR, cnt = _sc_aggregate(featL, featR, srcs, dsts)

  return _tc_combine(
      feat, aggL, aggR, cnt,
      W0, W_r0[:DH], W_r0[DH:], W_r1[:DH], W_r1[DH:], W_r2[:DH], W_r2[DH:],
      b0.reshape(1, D), b_r0.reshape(1, D), b_r1.reshape(1, D),
      b_r2.reshape(1, D))
